# Initial kernel scaffold; baseline (speedup 1.0000x reference)
import jax
import jax.numpy as jnp
from jax.experimental import pallas as pl


def kernel(token_ids, weight):
    raise NotImplementedError("write your pallas kernel here")


# speedup vs baseline: 1.8448x; 1.8448x over previous
import functools

import jax
import jax.numpy as jnp
from jax import lax
from jax.experimental import pallas as pl
from jax.experimental.pallas import tpu as pltpu
from jax.experimental.pallas import tpu_sc as plsc

_D = 64
_C = 256
_NBUF = 4


@functools.lru_cache(maxsize=None)
def _build(B):
    info = plsc.get_sparse_core_info()
    NC, NS = info.num_cores, info.num_subcores
    NW = NC * NS
    per_w = B // NW
    n_chunks = per_w // _C
    mesh = plsc.VectorSubcoreMesh(core_axis_name="c", subcore_axis_name="s")

    @functools.partial(
        pl.kernel,
        mesh=mesh,
        out_type=jax.ShapeDtypeStruct((B, _D), jnp.float32),
        scratch_types=[
            pltpu.VMEM((_NBUF, _C), jnp.int32),
            pltpu.VMEM((_NBUF, _C, _D), jnp.float32),
        ]
        + [pltpu.SemaphoreType.DMA] * _NBUF,
        compiler_params=pltpu.CompilerParams(use_tc_tiling_on_sc=False),
    )
    def grab(idx_hbm, table_hbm, out_hbm, idx_v, rows_v, *gsems):
        wid = lax.axis_index("s") * NC + lax.axis_index("c")
        base = wid * per_w

        for b in range(_NBUF):
            pltpu.sync_copy(idx_hbm.at[pl.ds(base + b * _C, _C)], idx_v.at[b])
            pltpu.async_copy(table_hbm.at[idx_v.at[b]], rows_v.at[b], gsems[b])

        @pl.loop(0, n_chunks, step=_NBUF)
        def _(g0):
            for b in range(_NBUF):
                g = g0 + b
                pltpu.make_async_copy(
                    table_hbm.at[idx_v.at[b]], rows_v.at[b], gsems[b]
                ).wait()
                pltpu.sync_copy(rows_v.at[b], out_hbm.at[pl.ds(base + g * _C, _C)])
                nxt = g + _NBUF

                @pl.when(nxt < n_chunks)
                def _():
                    pltpu.sync_copy(
                        idx_hbm.at[pl.ds(base + nxt * _C, _C)], idx_v.at[b]
                    )
                    pltpu.async_copy(
                        table_hbm.at[idx_v.at[b]], rows_v.at[b], gsems[b]
                    )

    return grab


def kernel(token_ids, weight):
    B, S = token_ids.shape
    idx_flat = token_ids.reshape(-1).astype(jnp.int32)
    out = _build(idx_flat.shape[0])(idx_flat, weight)
    return out.reshape(B, S, _D)


# speedup vs baseline: 1.8749x; 1.0164x over previous
import functools

import jax
import jax.numpy as jnp
from jax import lax
from jax.experimental import pallas as pl
from jax.experimental.pallas import tpu as pltpu
from jax.experimental.pallas import tpu_sc as plsc

_D = 64
_C = 256
_NBUF = 5
_LAG = 4


@functools.lru_cache(maxsize=None)
def _build(B):
    info = plsc.get_sparse_core_info()
    NC, NS = info.num_cores, info.num_subcores
    NW = NC * NS
    per_w = B // NW
    n_chunks = per_w // _C
    assert per_w % _C == 0 and n_chunks % _NBUF == 0
    mesh = plsc.VectorSubcoreMesh(core_axis_name="c", subcore_axis_name="s")

    @functools.partial(
        pl.kernel,
        mesh=mesh,
        out_type=jax.ShapeDtypeStruct((B, _D), jnp.float32),
        scratch_types=[
            pltpu.VMEM((per_w,), jnp.int32),
            pltpu.VMEM((_NBUF, _C, _D), jnp.float32),
        ]
        + [pltpu.SemaphoreType.DMA] * (2 * _NBUF),
        compiler_params=pltpu.CompilerParams(use_tc_tiling_on_sc=False),
    )
    def grab(idx_hbm, table_hbm, out_hbm, idx_v, rows_v, *sems):
        gsems, wsems = sems[:_NBUF], sems[_NBUF:]
        wid = lax.axis_index("s") * NC + lax.axis_index("c")
        base = wid * per_w

        pltpu.sync_copy(idx_hbm.at[pl.ds(base, per_w)], idx_v)

        def fire_gather(g, b):
            pltpu.async_copy(
                table_hbm.at[idx_v.at[pl.ds(g * _C, _C)]], rows_v.at[b], gsems[b]
            )

        def wait_write(g, b):
            pltpu.make_async_copy(
                rows_v.at[b], out_hbm.at[pl.ds(base + g * _C, _C)], wsems[b]
            ).wait()

        for g in range(_LAG):
            fire_gather(g, g % _NBUF)

        @pl.loop(0, n_chunks, step=_NBUF)
        def _(g0):
            for b in range(_NBUF):
                g = g0 + b
                gf = g + _LAG
                bf = (b + _LAG) % _NBUF

                @pl.when(gf < n_chunks)
                def _():
                    @pl.when(gf >= _NBUF)
                    def _():
                        wait_write(gf - _NBUF, bf)

                    fire_gather(gf, bf)

                pltpu.make_async_copy(
                    table_hbm.at[idx_v.at[pl.ds(g * _C, _C)]], rows_v.at[b], gsems[b]
                ).wait()
                pltpu.async_copy(
                    rows_v.at[b], out_hbm.at[pl.ds(base + g * _C, _C)], wsems[b]
                )

        for g in range(n_chunks - _NBUF, n_chunks):
            wait_write(g, g % _NBUF)

    return grab


def kernel(token_ids, weight):
    B, S = token_ids.shape
    idx_flat = token_ids.reshape(-1).astype(jnp.int32)
    out = _build(idx_flat.shape[0])(idx_flat, weight)
    return out.reshape(B, S, _D)
